# baseline (speedup 1.0000x reference)
import functools

import jax
import jax.numpy as jnp
from jax import lax
from jax.experimental import pallas as pl
from jax.experimental.pallas import tpu as pltpu
from jax.experimental.pallas import tpu_sc as plsc

B = 1024
L = 200
ED = 64
FD = 16
H = 64
R = B * L

_NC = 2
_NS = 16
_NW = _NC * _NS
_EDP = 128
_PER_W = R // _NW
_CHUNK = 400
_NCHUNK = _PER_W // _CHUNK


def _sc_gather(table128, idx_flat):
    mesh = plsc.VectorSubcoreMesh(core_axis_name="c", subcore_axis_name="s")

    @functools.partial(
        pl.kernel,
        out_type=jax.ShapeDtypeStruct((R, _EDP), jnp.float32),
        mesh=mesh,
        scratch_types=[
            pltpu.VMEM((_CHUNK,), jnp.int32),
            pltpu.VMEM((_CHUNK,), jnp.int32),
            pltpu.VMEM((_CHUNK, _EDP), jnp.float32),
            pltpu.VMEM((_CHUNK, _EDP), jnp.float32),
            pltpu.SemaphoreType.DMA,
            pltpu.SemaphoreType.DMA,
            pltpu.SemaphoreType.DMA,
            pltpu.SemaphoreType.DMA,
        ],
    )
    def gather_kernel(table_hbm, idx_hbm, out_hbm,
                      idx0, idx1, rows0, rows1, gs0, gs1, ws0, ws1):
        wid = lax.axis_index("s") * _NC + lax.axis_index("c")
        base = wid * _PER_W
        idx = (idx0, idx1)
        rows = (rows0, rows1)
        gs = (gs0, gs1)
        ws = (ws0, ws1)
        pend = [None, None]
        for c in range(_NCHUNK):
            b = c & 1
            off = base + c * _CHUNK
            if pend[b] is not None:
                pend[b].wait()
            pltpu.sync_copy(idx_hbm.at[pl.ds(off, _CHUNK)], idx[b])
            pltpu.async_copy(table_hbm.at[idx[b]], rows[b], gs[b]).wait()
            pend[b] = pltpu.async_copy(
                rows[b], out_hbm.at[pl.ds(off, _CHUNK)], ws[b])
        for p in pend:
            p.wait()

    return gather_kernel(table128, idx_flat)


def _gru_body(emb_f_ref, emb_b_ref, feat_f_ref, feat_b_ref, len_ref,
              wef, wff, whf, bif, bhf,
              web, wfb, whb, bib, bhb,
              out_f_ref, out_b_ref, hf_ref, hb_ref):
    s = pl.program_id(0)

    @pl.when(s == 0)
    def _init():
        hf_ref[...] = jnp.zeros_like(hf_ref)
        hb_ref[...] = jnp.zeros_like(hb_ref)

    def step(emb_ref, feat_ref, we, wf, wh, bi, bh, h_ref, out_ref, m):
        e = emb_ref[0]
        f = feat_ref[0]
        h = h_ref[...]
        gi = (jnp.dot(e, we[...], preferred_element_type=jnp.float32)
              + jnp.dot(f, wf[...], preferred_element_type=jnp.float32)
              + bi[...])
        gh = jnp.dot(h, wh[...], preferred_element_type=jnp.float32) + bh[...]
        r = jax.nn.sigmoid(gi[:, 0:H] + gh[:, 0:H])
        z = jax.nn.sigmoid(gi[:, H:2 * H] + gh[:, H:2 * H])
        n = jnp.tanh(gi[:, 2 * H:3 * H] + r * gh[:, 2 * H:3 * H])
        h_new = (1.0 - z) * n + z * h
        out = m * h_new
        h_ref[...] = out + (1.0 - m) * h
        out_ref[0] = out

    m_f = (len_ref[...] > s).astype(jnp.float32)
    m_b = (len_ref[...] > (L - 1 - s)).astype(jnp.float32)
    step(emb_f_ref, feat_f_ref, wef, wff, whf, bif, bhf, hf_ref, out_f_ref, m_f)
    step(emb_b_ref, feat_b_ref, web, wfb, whb, bib, bhb, hb_ref, out_b_ref, m_b)


def _gru_call(emb_tm, feat_tm, len_col,
              wef, wff, whf, bif, bhf,
              web, wfb, whb, bib, bhb):
    full = lambda shape: pl.BlockSpec(shape, lambda s: tuple(0 for _ in shape))
    fwd3 = pl.BlockSpec((1, B, _EDP), lambda s: (s, 0, 0))
    bwd3 = pl.BlockSpec((1, B, _EDP), lambda s: (L - 1 - s, 0, 0))
    fwdf = pl.BlockSpec((1, B, FD), lambda s: (s, 0, 0))
    bwdf = pl.BlockSpec((1, B, FD), lambda s: (L - 1 - s, 0, 0))
    outf = pl.BlockSpec((1, B, H), lambda s: (s, 0, 0))
    outb = pl.BlockSpec((1, B, H), lambda s: (L - 1 - s, 0, 0))
    return pl.pallas_call(
        _gru_body,
        grid=(L,),
        in_specs=[
            fwd3, bwd3, fwdf, bwdf, full((B, 1)),
            full((_EDP, 3 * H)), full((FD, 3 * H)), full((H, 3 * H)),
            full((1, 3 * H)), full((1, 3 * H)),
            full((_EDP, 3 * H)), full((FD, 3 * H)), full((H, 3 * H)),
            full((1, 3 * H)), full((1, 3 * H)),
        ],
        out_specs=[outf, outb],
        out_shape=[
            jax.ShapeDtypeStruct((L, B, H), jnp.float32),
            jax.ShapeDtypeStruct((L, B, H), jnp.float32),
        ],
        scratch_shapes=[
            pltpu.VMEM((B, H), jnp.float32),
            pltpu.VMEM((B, H), jnp.float32),
        ],
        compiler_params=pltpu.CompilerParams(
            dimension_semantics=("arbitrary",)),
    )(emb_tm, emb_tm, feat_tm, feat_tm, len_col,
      wef, wff, whf, bif, bhf,
      web, wfb, whb, bib, bhb)


def kernel(edgeIdOfPath, pathSegmentFeat, predFlow, emb_table,
           w_ih_f, w_hh_f, b_ih_f, b_hh_f,
           w_ih_b, w_hh_b, b_ih_b, b_hh_b):
    ids = edgeIdOfPath.astype(jnp.int32)
    len_col = jnp.sum((ids != 0).astype(jnp.int32), axis=1,
                      keepdims=True)
    idx_flat = ids.T.reshape(R)
    table128 = jnp.pad(emb_table, ((0, 0), (0, _EDP - ED)))
    emb_tm = _sc_gather(table128, idx_flat).reshape(L, B, _EDP)
    feat_tm = jnp.swapaxes(pathSegmentFeat, 0, 1)
    feat_tm = feat_tm.at[:, :, FD - 1].set(predFlow[None, :])

    pad_w = lambda w: jnp.pad(w[:, :ED].T, ((0, _EDP - ED), (0, 0)))
    out_f, out_b = _gru_call(
        emb_tm, feat_tm, len_col,
        pad_w(w_ih_f), w_ih_f[:, ED:].T, w_hh_f.T,
        b_ih_f.reshape(1, 3 * H), b_hh_f.reshape(1, 3 * H),
        pad_w(w_ih_b), w_ih_b[:, ED:].T, w_hh_b.T,
        b_ih_b.reshape(1, 3 * H), b_hh_b.reshape(1, 3 * H))

    return jnp.swapaxes(jnp.concatenate([out_f, out_b], axis=2), 0, 1)


# speedup vs baseline: 3.2880x; 3.2880x over previous
import functools

import jax
import jax.numpy as jnp
from jax import lax
from jax.experimental import pallas as pl
from jax.experimental.pallas import tpu as pltpu
from jax.experimental.pallas import tpu_sc as plsc

B = 1024
L = 200
ED = 64
FD = 16
H = 64
V = 100001
R = B * L

_NC = 2
_NS = 16
_NW = _NC * _NS
_PER_W = R // _NW
_CHUNK = 800
_NCHUNK = _PER_W // _CHUNK


def _sc_gather(table, idx_flat):
    mesh = plsc.VectorSubcoreMesh(core_axis_name="c", subcore_axis_name="s")

    @functools.partial(
        pl.kernel,
        out_type=jax.ShapeDtypeStruct((R, ED), jnp.float32),
        mesh=mesh,
        scratch_types=[
            pltpu.VMEM((_CHUNK,), jnp.int32),
            pltpu.VMEM((_CHUNK,), jnp.int32),
            pltpu.VMEM((_CHUNK, ED), jnp.float32),
            pltpu.VMEM((_CHUNK, ED), jnp.float32),
            pltpu.SemaphoreType.DMA,
            pltpu.SemaphoreType.DMA,
            pltpu.SemaphoreType.DMA,
            pltpu.SemaphoreType.DMA,
        ],
        compiler_params=pltpu.CompilerParams(use_tc_tiling_on_sc=False),
    )
    def gather_kernel(table_hbm, idx_hbm, out_hbm,
                      idx0, idx1, rows0, rows1, gs0, gs1, ws0, ws1):
        wid = lax.axis_index("s") * _NC + lax.axis_index("c")
        base = wid * _PER_W
        idx = (idx0, idx1)
        rows = (rows0, rows1)
        gs = (gs0, gs1)
        ws = (ws0, ws1)
        pend = [None, None]
        for c in range(_NCHUNK):
            b = c & 1
            off = base + c * _CHUNK
            if pend[b] is not None:
                pend[b].wait()
            pltpu.sync_copy(idx_hbm.at[pl.ds(off, _CHUNK)], idx[b])
            pltpu.async_copy(table_hbm.at[idx[b]], rows[b], gs[b]).wait()
            pend[b] = pltpu.async_copy(
                rows[b], out_hbm.at[pl.ds(off, _CHUNK)], ws[b])
        for p in pend:
            p.wait()

    return gather_kernel(table, idx_flat)


def _gru_body(emb_f_ref, emb_b_ref, feat_f_ref, feat_b_ref, len_ref,
              wef, wff, whf, bif, bhf,
              web, wfb, whb, bib, bhb,
              out_f_ref, out_b_ref, hf_ref, hb_ref):
    s = pl.program_id(0)

    @pl.when(s == 0)
    def _init():
        hf_ref[...] = jnp.zeros_like(hf_ref)
        hb_ref[...] = jnp.zeros_like(hb_ref)

    def step(emb_ref, feat_ref, we, wf, wh, bi, bh, h_ref, out_ref, m):
        e = emb_ref[0]
        f = feat_ref[0]
        h = h_ref[...]
        gi = (jnp.dot(e, we[...], preferred_element_type=jnp.float32)
              + jnp.dot(f, wf[...], preferred_element_type=jnp.float32)
              + bi[...])
        gh = jnp.dot(h, wh[...], preferred_element_type=jnp.float32) + bh[...]
        r = jax.nn.sigmoid(gi[:, 0:H] + gh[:, 0:H])
        z = jax.nn.sigmoid(gi[:, H:2 * H] + gh[:, H:2 * H])
        n = jnp.tanh(gi[:, 2 * H:3 * H] + r * gh[:, 2 * H:3 * H])
        h_new = (1.0 - z) * n + z * h
        out = m * h_new
        h_ref[...] = out + (1.0 - m) * h
        out_ref[0] = out

    m_f = (len_ref[...] > s).astype(jnp.float32)
    m_b = (len_ref[...] > (L - 1 - s)).astype(jnp.float32)
    step(emb_f_ref, feat_f_ref, wef, wff, whf, bif, bhf, hf_ref, out_f_ref, m_f)
    step(emb_b_ref, feat_b_ref, web, wfb, whb, bib, bhb, hb_ref, out_b_ref, m_b)


def _gru_call(emb_tm, feat_tm, len_col,
              wef, wff, whf, bif, bhf,
              web, wfb, whb, bib, bhb):
    full = lambda shape: pl.BlockSpec(shape, lambda s: tuple(0 for _ in shape))
    fwd3 = pl.BlockSpec((1, B, ED), lambda s: (s, 0, 0))
    bwd3 = pl.BlockSpec((1, B, ED), lambda s: (L - 1 - s, 0, 0))
    fwdf = pl.BlockSpec((1, B, FD), lambda s: (s, 0, 0))
    bwdf = pl.BlockSpec((1, B, FD), lambda s: (L - 1 - s, 0, 0))
    outf = pl.BlockSpec((1, B, H), lambda s: (s, 0, 0))
    outb = pl.BlockSpec((1, B, H), lambda s: (L - 1 - s, 0, 0))
    return pl.pallas_call(
        _gru_body,
        grid=(L,),
        in_specs=[
            fwd3, bwd3, fwdf, bwdf, full((B, 1)),
            full((ED, 3 * H)), full((FD, 3 * H)), full((H, 3 * H)),
            full((1, 3 * H)), full((1, 3 * H)),
            full((ED, 3 * H)), full((FD, 3 * H)), full((H, 3 * H)),
            full((1, 3 * H)), full((1, 3 * H)),
        ],
        out_specs=[outf, outb],
        out_shape=[
            jax.ShapeDtypeStruct((L, B, H), jnp.float32),
            jax.ShapeDtypeStruct((L, B, H), jnp.float32),
        ],
        scratch_shapes=[
            pltpu.VMEM((B, H), jnp.float32),
            pltpu.VMEM((B, H), jnp.float32),
        ],
        compiler_params=pltpu.CompilerParams(
            dimension_semantics=("arbitrary",)),
    )(emb_tm, emb_tm, feat_tm, feat_tm, len_col,
      wef, wff, whf, bif, bhf,
      web, wfb, whb, bib, bhb)


def kernel(edgeIdOfPath, pathSegmentFeat, predFlow, emb_table,
           w_ih_f, w_hh_f, b_ih_f, b_hh_f,
           w_ih_b, w_hh_b, b_ih_b, b_hh_b):
    ids = edgeIdOfPath.astype(jnp.int32)
    len_col = jnp.sum((ids != 0).astype(jnp.int32), axis=1,
                      keepdims=True)
    idx_flat = ids.T.reshape(R)
    idx_flat = jnp.where(idx_flat == 0,
                         jnp.arange(R, dtype=jnp.int32) % jnp.int32(V),
                         idx_flat)
    emb_tm = _sc_gather(emb_table, idx_flat).reshape(L, B, ED)
    feat_tm = jnp.swapaxes(pathSegmentFeat, 0, 1)
    feat_tm = feat_tm.at[:, :, FD - 1].set(predFlow[None, :])

    out_f, out_b = _gru_call(
        emb_tm, feat_tm, len_col,
        w_ih_f[:, :ED].T, w_ih_f[:, ED:].T, w_hh_f.T,
        b_ih_f.reshape(1, 3 * H), b_hh_f.reshape(1, 3 * H),
        w_ih_b[:, :ED].T, w_ih_b[:, ED:].T, w_hh_b.T,
        b_ih_b.reshape(1, 3 * H), b_hh_b.reshape(1, 3 * H))

    return jnp.swapaxes(jnp.concatenate([out_f, out_b], axis=2), 0, 1)


# speedup vs baseline: 3.5353x; 1.0752x over previous
import functools

import jax
import jax.numpy as jnp
from jax import lax
from jax.experimental import pallas as pl
from jax.experimental.pallas import tpu as pltpu
from jax.experimental.pallas import tpu_sc as plsc

B = 1024
L = 200
ED = 64
FD = 16
H = 64
V = 100001
R = B * L

_NC = 2
_NS = 16
_NW = _NC * _NS
_PER_W = R // _NW
_CHUNK = 800
_NCHUNK = _PER_W // _CHUNK


def _sc_gather(table, idx_flat):
    mesh = plsc.VectorSubcoreMesh(core_axis_name="c", subcore_axis_name="s")

    @functools.partial(
        pl.kernel,
        out_type=jax.ShapeDtypeStruct((R, ED), jnp.float32),
        mesh=mesh,
        scratch_types=[
            pltpu.VMEM((_CHUNK,), jnp.int32),
            pltpu.VMEM((_CHUNK,), jnp.int32),
            pltpu.VMEM((_CHUNK, ED), jnp.float32),
            pltpu.VMEM((_CHUNK, ED), jnp.float32),
            pltpu.SemaphoreType.DMA,
            pltpu.SemaphoreType.DMA,
            pltpu.SemaphoreType.DMA,
            pltpu.SemaphoreType.DMA,
        ],
        compiler_params=pltpu.CompilerParams(use_tc_tiling_on_sc=False),
    )
    def gather_kernel(table_hbm, idx_hbm, out_hbm,
                      idx0, idx1, rows0, rows1, gs0, gs1, ws0, ws1):
        wid = lax.axis_index("s") * _NC + lax.axis_index("c")
        base = wid * _PER_W
        idx = (idx0, idx1)
        rows = (rows0, rows1)
        gs = (gs0, gs1)
        ws = (ws0, ws1)
        pend = [None, None]
        for c in range(_NCHUNK):
            b = c & 1
            off = base + c * _CHUNK
            if pend[b] is not None:
                pend[b].wait()
            pltpu.sync_copy(idx_hbm.at[pl.ds(off, _CHUNK)], idx[b])
            pltpu.async_copy(table_hbm.at[idx[b]], rows[b], gs[b]).wait()
            pend[b] = pltpu.async_copy(
                rows[b], out_hbm.at[pl.ds(off, _CHUNK)], ws[b])
        for p in pend:
            p.wait()

    return gather_kernel(table, idx_flat)


def _gru_body(emb_f_ref, emb_b_ref, feat_f_ref, feat_b_ref, len_ref,
              wef, wff, whf, bif, bhf,
              web, wfb, whb, bib, bhb,
              out_f_ref, out_b_ref, hf_ref, hb_ref):
    s = pl.program_id(0)

    @pl.when(s == 0)
    def _init():
        hf_ref[...] = jnp.zeros_like(hf_ref)
        hb_ref[...] = jnp.zeros_like(hb_ref)

    def step(emb_ref, feat_ref, we, wf, wh, bi, bh, h_ref, out_ref, m):
        e = emb_ref[0]
        f = feat_ref[0]
        h = h_ref[...]
        gi = (jnp.dot(e, we[...], preferred_element_type=jnp.float32)
              + jnp.dot(f, wf[...], preferred_element_type=jnp.float32)
              + bi[...])
        gh = jnp.dot(h, wh[...], preferred_element_type=jnp.float32) + bh[...]
        a = gi + gh
        r = jax.nn.sigmoid(a[:, 0:H])
        z = jax.nn.sigmoid(a[:, H:2 * H])
        n = jnp.tanh(a[:, 2 * H:3 * H] + (r - 1.0) * gh[:, 2 * H:3 * H])
        h_new = n + z * (h - n)
        out = m * h_new
        h_ref[...] = h + m * (h_new - h)
        out_ref[0] = out

    m_f = (len_ref[...] > s).astype(jnp.float32)
    m_b = (len_ref[...] > (L - 1 - s)).astype(jnp.float32)
    step(emb_f_ref, feat_f_ref, wef, wff, whf, bif, bhf, hf_ref, out_f_ref, m_f)
    step(emb_b_ref, feat_b_ref, web, wfb, whb, bib, bhb, hb_ref, out_b_ref, m_b)


def _gru_call(emb_tm, feat_tm, len_col,
              wef, wff, whf, bif, bhf,
              web, wfb, whb, bib, bhb):
    full = lambda shape: pl.BlockSpec(shape, lambda s: tuple(0 for _ in shape))
    fwd3 = pl.BlockSpec((1, B, ED), lambda s: (s, 0, 0))
    bwd3 = pl.BlockSpec((1, B, ED), lambda s: (L - 1 - s, 0, 0))
    fwdf = pl.BlockSpec((1, B, FD), lambda s: (s, 0, 0))
    bwdf = pl.BlockSpec((1, B, FD), lambda s: (L - 1 - s, 0, 0))
    outf = pl.BlockSpec((1, B, H), lambda s: (s, 0, 0))
    outb = pl.BlockSpec((1, B, H), lambda s: (L - 1 - s, 0, 0))
    return pl.pallas_call(
        _gru_body,
        grid=(L,),
        in_specs=[
            fwd3, bwd3, fwdf, bwdf, full((B, H)),
            full((ED, 3 * H)), full((FD, 3 * H)), full((H, 3 * H)),
            full((1, 3 * H)), full((1, 3 * H)),
            full((ED, 3 * H)), full((FD, 3 * H)), full((H, 3 * H)),
            full((1, 3 * H)), full((1, 3 * H)),
        ],
        out_specs=[outf, outb],
        out_shape=[
            jax.ShapeDtypeStruct((L, B, H), jnp.float32),
            jax.ShapeDtypeStruct((L, B, H), jnp.float32),
        ],
        scratch_shapes=[
            pltpu.VMEM((B, H), jnp.float32),
            pltpu.VMEM((B, H), jnp.float32),
        ],
        compiler_params=pltpu.CompilerParams(
            dimension_semantics=("arbitrary",)),
    )(emb_tm, emb_tm, feat_tm, feat_tm, len_col,
      wef, wff, whf, bif, bhf,
      web, wfb, whb, bib, bhb)


def kernel(edgeIdOfPath, pathSegmentFeat, predFlow, emb_table,
           w_ih_f, w_hh_f, b_ih_f, b_hh_f,
           w_ih_b, w_hh_b, b_ih_b, b_hh_b):
    ids = edgeIdOfPath.astype(jnp.int32)
    len_col = jnp.broadcast_to(
        jnp.sum((ids != 0).astype(jnp.int32), axis=1, keepdims=True),
        (B, H))
    idx_flat = ids.T.reshape(R)
    idx_flat = jnp.where(idx_flat == 0,
                         jnp.arange(R, dtype=jnp.int32) % jnp.int32(V),
                         idx_flat)
    emb_tm = _sc_gather(emb_table, idx_flat).reshape(L, B, ED)
    feat_tm = jnp.swapaxes(pathSegmentFeat, 0, 1)
    feat_tm = feat_tm.at[:, :, FD - 1].set(predFlow[None, :])

    out_f, out_b = _gru_call(
        emb_tm, feat_tm, len_col,
        w_ih_f[:, :ED].T, w_ih_f[:, ED:].T, w_hh_f.T,
        b_ih_f.reshape(1, 3 * H), b_hh_f.reshape(1, 3 * H),
        w_ih_b[:, :ED].T, w_ih_b[:, ED:].T, w_hh_b.T,
        b_ih_b.reshape(1, 3 * H), b_hh_b.reshape(1, 3 * H))

    return jnp.swapaxes(jnp.concatenate([out_f, out_b], axis=2), 0, 1)


# speedup vs baseline: 3.9254x; 1.1103x over previous
import functools

import jax
import jax.numpy as jnp
from jax import lax
from jax.experimental import pallas as pl
from jax.experimental.pallas import tpu as pltpu
from jax.experimental.pallas import tpu_sc as plsc

B = 1024
L = 200
ED = 64
FD = 16
H = 64
V = 100001
R = B * L

_NC = 2
_NS = 16
_NW = _NC * _NS
_PER_W = R // _NW
_CHUNK = 800
_NCHUNK = _PER_W // _CHUNK


def _sc_gather(table, idx_flat):
    mesh = plsc.VectorSubcoreMesh(core_axis_name="c", subcore_axis_name="s")

    @functools.partial(
        pl.kernel,
        out_type=jax.ShapeDtypeStruct((R, ED), jnp.float32),
        mesh=mesh,
        scratch_types=[
            pltpu.VMEM((_CHUNK,), jnp.int32),
            pltpu.VMEM((_CHUNK,), jnp.int32),
            pltpu.VMEM((_CHUNK, ED), jnp.float32),
            pltpu.VMEM((_CHUNK, ED), jnp.float32),
            pltpu.SemaphoreType.DMA,
            pltpu.SemaphoreType.DMA,
            pltpu.SemaphoreType.DMA,
            pltpu.SemaphoreType.DMA,
        ],
        compiler_params=pltpu.CompilerParams(use_tc_tiling_on_sc=False),
    )
    def gather_kernel(table_hbm, idx_hbm, out_hbm,
                      idx0, idx1, rows0, rows1, gs0, gs1, ws0, ws1):
        wid = lax.axis_index("s") * _NC + lax.axis_index("c")
        base = wid * _PER_W
        idx = (idx0, idx1)
        rows = (rows0, rows1)
        gs = (gs0, gs1)
        ws = (ws0, ws1)
        pend = [None, None]
        for c in range(_NCHUNK):
            b = c & 1
            off = base + c * _CHUNK
            if pend[b] is not None:
                pend[b].wait()
            pltpu.sync_copy(idx_hbm.at[pl.ds(off, _CHUNK)], idx[b])
            pltpu.async_copy(table_hbm.at[idx[b]], rows[b], gs[b]).wait()
            pend[b] = pltpu.async_copy(
                rows[b], out_hbm.at[pl.ds(off, _CHUNK)], ws[b])
        for p in pend:
            p.wait()

    return gather_kernel(table, idx_flat)


_T = 8
_NT = L // _T


def _gru_body(emb_f_ref, emb_b_ref, feat_f_ref, feat_b_ref, len_ref,
              wef, wff, whf, bif, bhf,
              web, wfb, whb, bib, bhb,
              out_f_ref, out_b_ref, hf_ref, hb_ref):
    s = pl.program_id(0)

    @pl.when(s == 0)
    def _init():
        hf_ref[...] = jnp.zeros_like(hf_ref)
        hb_ref[...] = jnp.zeros_like(hb_ref)

    def step(emb_ref, feat_ref, we, wf, wh, bi, bh, h_ref, out_ref, m, k):
        e = emb_ref[k]
        f = feat_ref[k]
        h = h_ref[...]
        gi = (jnp.dot(e, we[...], preferred_element_type=jnp.float32)
              + jnp.dot(f, wf[...], preferred_element_type=jnp.float32)
              + bi[...])
        gh = jnp.dot(h, wh[...], preferred_element_type=jnp.float32) + bh[...]
        a = gi + gh
        r = jax.nn.sigmoid(a[:, 0:H])
        z = jax.nn.sigmoid(a[:, H:2 * H])
        n = jnp.tanh(a[:, 2 * H:3 * H] + (r - 1.0) * gh[:, 2 * H:3 * H])
        h_new = n + z * (h - n)
        out = m * h_new
        h_ref[...] = h + m * (h_new - h)
        out_ref[k] = out

    for k in range(_T):
        t_f = s * _T + k
        m_f = (len_ref[...] > t_f).astype(jnp.float32)
        m_b = (len_ref[...] > (L - 1 - t_f)).astype(jnp.float32)
        step(emb_f_ref, feat_f_ref, wef, wff, whf, bif, bhf,
             hf_ref, out_f_ref, m_f, k)
        step(emb_b_ref, feat_b_ref, web, wfb, whb, bib, bhb,
             hb_ref, out_b_ref, m_b, _T - 1 - k)


def _gru_call(emb_tm, feat_tm, len_col,
              wef, wff, whf, bif, bhf,
              web, wfb, whb, bib, bhb):
    full = lambda shape: pl.BlockSpec(shape, lambda s: tuple(0 for _ in shape))
    fwd3 = pl.BlockSpec((_T, B, ED), lambda s: (s, 0, 0))
    bwd3 = pl.BlockSpec((_T, B, ED), lambda s: (_NT - 1 - s, 0, 0))
    fwdf = pl.BlockSpec((_T, B, FD), lambda s: (s, 0, 0))
    bwdf = pl.BlockSpec((_T, B, FD), lambda s: (_NT - 1 - s, 0, 0))
    outf = pl.BlockSpec((_T, B, H), lambda s: (s, 0, 0))
    outb = pl.BlockSpec((_T, B, H), lambda s: (_NT - 1 - s, 0, 0))
    return pl.pallas_call(
        _gru_body,
        grid=(_NT,),
        in_specs=[
            fwd3, bwd3, fwdf, bwdf, full((B, H)),
            full((ED, 3 * H)), full((FD, 3 * H)), full((H, 3 * H)),
            full((1, 3 * H)), full((1, 3 * H)),
            full((ED, 3 * H)), full((FD, 3 * H)), full((H, 3 * H)),
            full((1, 3 * H)), full((1, 3 * H)),
        ],
        out_specs=[outf, outb],
        out_shape=[
            jax.ShapeDtypeStruct((L, B, H), jnp.float32),
            jax.ShapeDtypeStruct((L, B, H), jnp.float32),
        ],
        scratch_shapes=[
            pltpu.VMEM((B, H), jnp.float32),
            pltpu.VMEM((B, H), jnp.float32),
        ],
        compiler_params=pltpu.CompilerParams(
            dimension_semantics=("arbitrary",)),
    )(emb_tm, emb_tm, feat_tm, feat_tm, len_col,
      wef, wff, whf, bif, bhf,
      web, wfb, whb, bib, bhb)


def kernel(edgeIdOfPath, pathSegmentFeat, predFlow, emb_table,
           w_ih_f, w_hh_f, b_ih_f, b_hh_f,
           w_ih_b, w_hh_b, b_ih_b, b_hh_b):
    ids = edgeIdOfPath.astype(jnp.int32)
    len_col = jnp.broadcast_to(
        jnp.sum((ids != 0).astype(jnp.int32), axis=1, keepdims=True),
        (B, H))
    idx_flat = ids.T.reshape(R)
    idx_flat = jnp.where(idx_flat == 0,
                         jnp.arange(R, dtype=jnp.int32) % jnp.int32(V),
                         idx_flat)
    emb_tm = _sc_gather(emb_table, idx_flat).reshape(L, B, ED)
    feat_tm = jnp.swapaxes(pathSegmentFeat, 0, 1)
    feat_tm = feat_tm.at[:, :, FD - 1].set(predFlow[None, :])

    out_f, out_b = _gru_call(
        emb_tm, feat_tm, len_col,
        w_ih_f[:, :ED].T, w_ih_f[:, ED:].T, w_hh_f.T,
        b_ih_f.reshape(1, 3 * H), b_hh_f.reshape(1, 3 * H),
        w_ih_b[:, :ED].T, w_ih_b[:, ED:].T, w_hh_b.T,
        b_ih_b.reshape(1, 3 * H), b_hh_b.reshape(1, 3 * H))

    return jnp.swapaxes(jnp.concatenate([out_f, out_b], axis=2), 0, 1)


# speedup vs baseline: 3.9272x; 1.0005x over previous
import functools

import jax
import jax.numpy as jnp
from jax import lax
from jax.experimental import pallas as pl
from jax.experimental.pallas import tpu as pltpu
from jax.experimental.pallas import tpu_sc as plsc

B = 1024
L = 200
ED = 64
FD = 16
H = 64
V = 100001
R = B * L

_NC = 2
_NS = 16
_NW = _NC * _NS
_PER_W = R // _NW
_CHUNK = 800
_NCHUNK = _PER_W // _CHUNK


def _sc_gather(table, idx_flat):
    mesh = plsc.VectorSubcoreMesh(core_axis_name="c", subcore_axis_name="s")

    @functools.partial(
        pl.kernel,
        out_type=jax.ShapeDtypeStruct((R, ED), jnp.float32),
        mesh=mesh,
        scratch_types=[
            pltpu.VMEM((_CHUNK,), jnp.int32),
            pltpu.VMEM((_CHUNK,), jnp.int32),
            pltpu.VMEM((_CHUNK, ED), jnp.float32),
            pltpu.VMEM((_CHUNK, ED), jnp.float32),
            pltpu.SemaphoreType.DMA,
            pltpu.SemaphoreType.DMA,
            pltpu.SemaphoreType.DMA,
            pltpu.SemaphoreType.DMA,
        ],
        compiler_params=pltpu.CompilerParams(use_tc_tiling_on_sc=False),
    )
    def gather_kernel(table_hbm, idx_hbm, out_hbm,
                      idx0, idx1, rows0, rows1, gs0, gs1, ws0, ws1):
        wid = lax.axis_index("s") * _NC + lax.axis_index("c")
        base = wid * _PER_W
        idx = (idx0, idx1)
        rows = (rows0, rows1)
        gs = (gs0, gs1)
        ws = (ws0, ws1)
        pend = [None, None]
        for c in range(_NCHUNK):
            b = c & 1
            off = base + c * _CHUNK
            if pend[b] is not None:
                pend[b].wait()
            pltpu.sync_copy(idx_hbm.at[pl.ds(off, _CHUNK)], idx[b])
            pltpu.async_copy(table_hbm.at[idx[b]], rows[b], gs[b]).wait()
            pend[b] = pltpu.async_copy(
                rows[b], out_hbm.at[pl.ds(off, _CHUNK)], ws[b])
        for p in pend:
            p.wait()

    return gather_kernel(table, idx_flat)


_T = 8
_NT = L // _T


def _gru_body(emb_f_ref, emb_b_ref, feat_f_ref, feat_b_ref, len_ref,
              wef, wff, whf, bif, bhf,
              web, wfb, whb, bib, bhb,
              out_f_ref, out_b_ref, hf_ref, hb_ref):
    s = pl.program_id(0)

    @pl.when(s == 0)
    def _init():
        hf_ref[...] = jnp.zeros_like(hf_ref)
        hb_ref[...] = jnp.zeros_like(hb_ref)

    def step(emb_ref, feat_ref, we, wf, wh, bi, bh, h_ref, out_ref, m, k):
        e = emb_ref[k]
        f = feat_ref[k]
        h = h_ref[...]
        gi = (jnp.dot(e, we[...], preferred_element_type=jnp.float32)
              + jnp.dot(f, wf[...], preferred_element_type=jnp.float32)
              + bi[...])
        gh = jnp.dot(h, wh[...], preferred_element_type=jnp.float32) + bh[...]
        a = gi + gh
        r = jax.nn.sigmoid(a[:, 0:H])
        z = jax.nn.sigmoid(a[:, H:2 * H])
        n = jnp.tanh(a[:, 2 * H:3 * H] + (r - 1.0) * gh[:, 2 * H:3 * H])
        h_new = n + z * (h - n)
        out = m * h_new
        h_ref[...] = h + m * (h_new - h)
        out_ref[k] = out

    for k in range(_T):
        t_f = s * _T + k
        m_f = (len_ref[...] > t_f).astype(jnp.float32)
        m_b = (len_ref[...] > (L - 1 - t_f)).astype(jnp.float32)
        step(emb_f_ref, feat_f_ref, wef, wff, whf, bif, bhf,
             hf_ref, out_f_ref, m_f, k)
        step(emb_b_ref, feat_b_ref, web, wfb, whb, bib, bhb,
             hb_ref, out_b_ref, m_b, _T - 1 - k)


def _gru_call(emb_tm, feat_tm, len_col,
              wef, wff, whf, bif, bhf,
              web, wfb, whb, bib, bhb):
    full = lambda shape: pl.BlockSpec(shape, lambda s: tuple(0 for _ in shape))
    fwd3 = pl.BlockSpec((_T, B, ED), lambda s: (s, 0, 0))
    bwd3 = pl.BlockSpec((_T, B, ED), lambda s: (_NT - 1 - s, 0, 0))
    fwdf = pl.BlockSpec((_T, B, FD), lambda s: (s, 0, 0))
    bwdf = pl.BlockSpec((_T, B, FD), lambda s: (_NT - 1 - s, 0, 0))
    outf = pl.BlockSpec((_T, B, H), lambda s: (s, 0, 0))
    outb = pl.BlockSpec((_T, B, H), lambda s: (_NT - 1 - s, 0, 0))
    return pl.pallas_call(
        _gru_body,
        grid=(_NT,),
        in_specs=[
            fwd3, bwd3, fwdf, bwdf, full((B, H)),
            full((ED, 3 * H)), full((FD, 3 * H)), full((H, 3 * H)),
            full((1, 3 * H)), full((1, 3 * H)),
            full((ED, 3 * H)), full((FD, 3 * H)), full((H, 3 * H)),
            full((1, 3 * H)), full((1, 3 * H)),
        ],
        out_specs=[outf, outb],
        out_shape=[
            jax.ShapeDtypeStruct((L, B, H), jnp.float32),
            jax.ShapeDtypeStruct((L, B, H), jnp.float32),
        ],
        scratch_shapes=[
            pltpu.VMEM((B, H), jnp.float32),
            pltpu.VMEM((B, H), jnp.float32),
        ],
        compiler_params=pltpu.CompilerParams(
            dimension_semantics=("arbitrary",)),
    )(emb_tm, emb_tm, feat_tm, feat_tm, len_col,
      wef, wff, whf, bif, bhf,
      web, wfb, whb, bib, bhb)


def kernel(edgeIdOfPath, pathSegmentFeat, predFlow, emb_table,
           w_ih_f, w_hh_f, b_ih_f, b_hh_f,
           w_ih_b, w_hh_b, b_ih_b, b_hh_b):
    ids = edgeIdOfPath.astype(jnp.int32)
    len_col = jnp.broadcast_to(
        jnp.sum((ids != 0).astype(jnp.int32), axis=1, keepdims=True),
        (B, H))
    idx_flat = ids.T.reshape(R)
    idx_flat = jnp.where(idx_flat == 0,
                         jnp.arange(R, dtype=jnp.int32) % jnp.int32(V),
                         idx_flat)
    emb_tm = _sc_gather(emb_table, idx_flat).reshape(L, B, ED)
    feat_tm = jnp.swapaxes(pathSegmentFeat, 0, 1)
    feat_tm = feat_tm.at[:, :, FD - 1].set(predFlow[None, :])

    out_f, out_b = _gru_call(
        emb_tm, feat_tm, len_col,
        w_ih_f[:, :ED].T, w_ih_f[:, ED:].T, w_hh_f.T,
        b_ih_f[None, :], b_hh_f[None, :],
        w_ih_b[:, :ED].T, w_ih_b[:, ED:].T, w_hh_b.T,
        b_ih_b[None, :], b_hh_b[None, :])

    return jnp.swapaxes(jnp.concatenate([out_f, out_b], axis=2), 0, 1)
